# baseline (speedup 1.0000x reference)
import jax
import jax.numpy as jnp
from jax.experimental import pallas as pl
from jax.experimental.pallas import tpu as pltpu

_BH = 16


def _pce_block(logits_ref, targets_ref, nll_sum_ref, count_ref):
    step = pl.program_id(0) * pl.num_programs(1) + pl.program_id(1)

    @pl.when(step == 0)
    def _init():
        nll_sum_ref[0, 0] = 0.0
        count_ref[0, 0] = 0.0

    t = targets_ref[0]
    C = logits_ref.shape[1]
    bh, W = t.shape

    valid = t != -1
    t_safe = jnp.where(valid, t, 0)

    def pass_a(i, carry):
        m, picked = carry
        x_c = logits_ref[0, i]
        m = jnp.maximum(m, x_c)
        picked = jnp.where(t_safe == i, x_c, picked)
        return m, picked

    neg_inf = jnp.full((bh, W), -jnp.inf, jnp.float32)
    zeros = jnp.zeros((bh, W), jnp.float32)
    m, picked = jax.lax.fori_loop(0, C, pass_a, (neg_inf, zeros))

    def pass_b(i, e):
        return e + jnp.exp(logits_ref[0, i] - m)

    e = jax.lax.fori_loop(0, C, pass_b, zeros)

    nll = (m + jnp.log(e) - picked) * valid.astype(jnp.float32)
    nll_sum_ref[0, 0] += jnp.sum(nll)
    count_ref[0, 0] += jnp.sum(valid.astype(jnp.float32))


@jax.jit
def kernel(logits, targets):
    B, C, H, W = logits.shape
    grid = (B, H // _BH)
    nll_sum, count = pl.pallas_call(
        _pce_block,
        grid=grid,
        in_specs=[
            pl.BlockSpec((1, C, _BH, W), lambda b, j: (b, 0, j, 0)),
            pl.BlockSpec((1, _BH, W), lambda b, j: (b, j, 0)),
        ],
        out_specs=[
            pl.BlockSpec(memory_space=pltpu.SMEM, block_shape=(1, 1),
                         index_map=lambda b, j: (0, 0)),
            pl.BlockSpec(memory_space=pltpu.SMEM, block_shape=(1, 1),
                         index_map=lambda b, j: (0, 0)),
        ],
        out_shape=[
            jax.ShapeDtypeStruct((1, 1), jnp.float32),
            jax.ShapeDtypeStruct((1, 1), jnp.float32),
        ],
    )(logits, targets)
    nll_sum = nll_sum[0, 0]
    count = count[0, 0]
    loss = nll_sum / jnp.maximum(count, 1.0)
    return jnp.where(count == 0.0, jnp.float32(0.0), loss)


# speedup vs baseline: 1.1466x; 1.1466x over previous
import jax
import jax.numpy as jnp
from jax.experimental import pallas as pl
from jax.experimental.pallas import tpu as pltpu

_BH = 8


def _pce_block(logits_ref, targets_ref, nll_sum_ref, count_ref):
    step = pl.program_id(0) * pl.num_programs(1) + pl.program_id(1)

    @pl.when(step == 0)
    def _init():
        nll_sum_ref[0, 0] = 0.0
        count_ref[0, 0] = 0.0

    x = logits_ref[0]
    t = targets_ref[0]

    valid = t != -1
    t_safe = jnp.where(valid, t, 0)

    m = jnp.max(x, axis=0)
    cls = jax.lax.broadcasted_iota(jnp.int32, x.shape, 0)
    e = jnp.sum(jnp.exp(x - m[None]), axis=0)
    picked = jnp.sum(jnp.where(cls == t_safe[None], x, 0.0), axis=0)

    nll = (m + jnp.log(e) - picked) * valid.astype(jnp.float32)
    nll_sum_ref[0, 0] += jnp.sum(nll)
    count_ref[0, 0] += jnp.sum(valid.astype(jnp.float32))


@jax.jit
def kernel(logits, targets):
    B, C, H, W = logits.shape
    grid = (B, H // _BH)
    nll_sum, count = pl.pallas_call(
        _pce_block,
        grid=grid,
        in_specs=[
            pl.BlockSpec((1, C, _BH, W), lambda b, j: (b, 0, j, 0)),
            pl.BlockSpec((1, _BH, W), lambda b, j: (b, j, 0)),
        ],
        out_specs=[
            pl.BlockSpec(memory_space=pltpu.SMEM, block_shape=(1, 1),
                         index_map=lambda b, j: (0, 0)),
            pl.BlockSpec(memory_space=pltpu.SMEM, block_shape=(1, 1),
                         index_map=lambda b, j: (0, 0)),
        ],
        out_shape=[
            jax.ShapeDtypeStruct((1, 1), jnp.float32),
            jax.ShapeDtypeStruct((1, 1), jnp.float32),
        ],
    )(logits, targets)
    nll_sum = nll_sum[0, 0]
    count = count[0, 0]
    loss = nll_sum / jnp.maximum(count, 1.0)
    return jnp.where(count == 0.0, jnp.float32(0.0), loss)


# speedup vs baseline: 2.0604x; 1.7970x over previous
import jax
import jax.numpy as jnp
from jax.experimental import pallas as pl
from jax.experimental.pallas import tpu as pltpu

_BH = 64
_BC = 8


def _pce_block(logits_ref, targets_ref, nll_sum_ref, count_ref):
    step = pl.program_id(0) * pl.num_programs(1) + pl.program_id(1)

    @pl.when(step == 0)
    def _init():
        nll_sum_ref[0, 0] = 0.0
        count_ref[0, 0] = 0.0

    W = logits_ref.shape[3]
    nll_acc = jnp.zeros((_BC, W), jnp.float32)
    cnt_acc = jnp.zeros((_BC, W), jnp.float32)
    for k in range(_BH // _BC):
        x = logits_ref[0, :, pl.ds(k * _BC, _BC), :]
        t = targets_ref[0, pl.ds(k * _BC, _BC), :]

        valid = t != -1
        t_safe = jnp.where(valid, t, 0)

        m = jnp.max(x, axis=0)
        cls = jax.lax.broadcasted_iota(jnp.int32, x.shape, 0)
        e = jnp.sum(jnp.exp(x - m[None]), axis=0)
        picked = jnp.sum(jnp.where(cls == t_safe[None], x, 0.0), axis=0)

        vf = valid.astype(jnp.float32)
        nll_acc += (m + jnp.log(e) - picked) * vf
        cnt_acc += vf

    nll_sum_ref[0, 0] += jnp.sum(nll_acc)
    count_ref[0, 0] += jnp.sum(cnt_acc)


@jax.jit
def kernel(logits, targets):
    B, C, H, W = logits.shape
    grid = (B, H // _BH)
    nll_sum, count = pl.pallas_call(
        _pce_block,
        grid=grid,
        in_specs=[
            pl.BlockSpec((1, C, _BH, W), lambda b, j: (b, 0, j, 0)),
            pl.BlockSpec((1, _BH, W), lambda b, j: (b, j, 0)),
        ],
        out_specs=[
            pl.BlockSpec(memory_space=pltpu.SMEM, block_shape=(1, 1),
                         index_map=lambda b, j: (0, 0)),
            pl.BlockSpec(memory_space=pltpu.SMEM, block_shape=(1, 1),
                         index_map=lambda b, j: (0, 0)),
        ],
        out_shape=[
            jax.ShapeDtypeStruct((1, 1), jnp.float32),
            jax.ShapeDtypeStruct((1, 1), jnp.float32),
        ],
    )(logits, targets)
    nll_sum = nll_sum[0, 0]
    count = count[0, 0]
    loss = nll_sum / jnp.maximum(count, 1.0)
    return jnp.where(count == 0.0, jnp.float32(0.0), loss)
